# Initial kernel scaffold; baseline (speedup 1.0000x reference)
import jax
import jax.numpy as jnp
from jax.experimental import pallas as pl


def kernel(x, edge_index, W1, b1, W2, b2, fcW, fcb, a):
    raise NotImplementedError("write your pallas kernel here")


# speedup vs baseline: 25.5017x; 25.5017x over previous
import functools

import jax
import jax.numpy as jnp
from jax import lax
from jax.experimental import pallas as pl
from jax.experimental.pallas import tpu as pltpu
from jax.experimental.pallas import tpu_sc as plsc

N_NODES = 10000
N_EDGES = 320000
NC = 2
NS = 16
NW = NC * NS
EPW = N_EDGES // NW
K = 80
NCHUNK = EPW // K
NPAD = 10240
RPT = NPAD // NS


def _zero_rows(buf, nrows, ncol16):
    def body(r, _):
        for j in range(ncol16):
            buf[r, pl.ds(j * 16, 16)] = jnp.zeros((16,), jnp.float32)
        return 0
    lax.fori_loop(0, nrows, body, 0)


def _copy_slab(src_buf, dst_ref, base):
    for kk in range(RPT // K):
        pltpu.sync_copy(src_buf, dst_ref.at[pl.ds(base + kk * K, K)])


def _sc_degree(dst3):
    mesh = plsc.VectorSubcoreMesh(core_axis_name="c", subcore_axis_name="s")

    @functools.partial(
        pl.kernel,
        out_type=jax.ShapeDtypeStruct((NC, NPAD, 16), jnp.float32),
        mesh=mesh,
        compiler_params=pltpu.CompilerParams(use_tc_tiling_on_sc=False),
        scratch_types=[
            pltpu.VMEM((NCHUNK, K), jnp.int32),
            pltpu.VMEM((K, 16), jnp.float32),
            pltpu.VMEM_SHARED((NPAD, 16), jnp.float32),
        ],
    )
    def deg_kernel(dst_hbm, out_hbm, idx_v, buf_v, acc_sh):
        c = lax.axis_index("c")
        s = lax.axis_index("s")
        wid = c * NS + s
        tbase = s * RPT
        _zero_rows(buf_v, K, 1)
        _copy_slab(buf_v, acc_sh, tbase)

        def ones_row(r, _):
            buf_v[r, :] = jnp.ones((16,), jnp.float32)
            return 0
        lax.fori_loop(0, K, ones_row, 0)
        pltpu.sync_copy(dst_hbm.at[wid], idx_v)
        plsc.subcore_barrier()

        def chunk(i, _):
            pltpu.sync_copy(buf_v, acc_sh.at[idx_v.at[i]], add=True)
            return 0
        lax.fori_loop(0, NCHUNK, chunk, 0)
        plsc.subcore_barrier()
        pltpu.sync_copy(acc_sh.at[pl.ds(tbase, RPT)],
                        out_hbm.at[c, pl.ds(tbase, RPT)])

    return deg_kernel(dst3)


def _sc_aggregate(src3, dst3, y, d):
    mesh = plsc.VectorSubcoreMesh(core_axis_name="c", subcore_axis_name="s")
    staged = d < 128
    scratch = [
        pltpu.VMEM((NCHUNK, K), jnp.int32),
        pltpu.VMEM((NCHUNK, K), jnp.int32),
        pltpu.VMEM((K, d), jnp.float32),
        pltpu.VMEM_SHARED((NPAD, d), jnp.float32),
        pltpu.SemaphoreType.DMA,
    ]
    if staged:
        scratch.insert(4, pltpu.VMEM_SHARED((NPAD, d), jnp.float32))
        y = jnp.concatenate(
            [y, jnp.zeros((NPAD - y.shape[0], d), jnp.float32)])

    @functools.partial(
        pl.kernel,
        out_type=jax.ShapeDtypeStruct((NC, NPAD, d), jnp.float32),
        mesh=mesh,
        compiler_params=pltpu.CompilerParams(use_tc_tiling_on_sc=False),
        scratch_types=scratch,
    )
    def agg_kernel(src_hbm, dst_hbm, y_hbm, out_hbm, src_v, dst_v, rows_v,
                   acc_sh, *rest):
        if staged:
            y_sh, sem = rest
        else:
            (sem,) = rest
        c = lax.axis_index("c")
        s = lax.axis_index("s")
        wid = c * NS + s
        tbase = s * RPT
        _zero_rows(rows_v, K, d // 16)
        _copy_slab(rows_v, acc_sh, tbase)
        pltpu.sync_copy(src_hbm.at[wid], src_v)
        pltpu.sync_copy(dst_hbm.at[wid], dst_v)
        if staged:
            pltpu.sync_copy(y_hbm.at[pl.ds(tbase, RPT)],
                            y_sh.at[pl.ds(tbase, RPT)])
            table = y_sh
        else:
            table = y_hbm
        plsc.subcore_barrier()

        def chunk(i, _):
            pltpu.async_copy(table.at[src_v.at[i]], rows_v, sem).wait()
            pltpu.sync_copy(rows_v, acc_sh.at[dst_v.at[i]], add=True)
            return 0
        lax.fori_loop(0, NCHUNK, chunk, 0)
        plsc.subcore_barrier()
        pltpu.sync_copy(acc_sh.at[pl.ds(tbase, RPT)],
                        out_hbm.at[c, pl.ds(tbase, RPT)])

    return agg_kernel(src3, dst3, y)


def _dinv_block(dp0, dp1):
    return lax.rsqrt(dp0[:, 0:1] + dp1[:, 0:1] + 1.0)


def _tc_layer1_pre(x, W1, degP, bm=1000):
    m, din = x.shape
    h = W1.shape[1]

    def body(x_ref, w_ref, dp_ref, y_ref):
        dinv = _dinv_block(dp_ref[0], dp_ref[1])
        xw = jnp.dot(x_ref[...], w_ref[...],
                     preferred_element_type=jnp.float32)
        y_ref[...] = xw * dinv

    return pl.pallas_call(
        body,
        grid=(m // bm,),
        in_specs=[
            pl.BlockSpec((bm, din), lambda i: (i, 0)),
            pl.BlockSpec((din, h), lambda i: (0, 0)),
            pl.BlockSpec((2, bm, 16), lambda i: (0, i, 0)),
        ],
        out_specs=pl.BlockSpec((bm, h), lambda i: (i, 0)),
        out_shape=jax.ShapeDtypeStruct((m, h), jnp.float32),
    )(x, W1, degP)


def _tc_layer1_post(z0, z1, y1, degP, W2, b1, a, bm=1000):
    m, h = y1.shape
    h2 = W2.shape[1]

    def body(z0_ref, z1_ref, y1_ref, dp_ref, w_ref, b_ref, a_ref, y2_ref):
        dinv = _dinv_block(dp_ref[0], dp_ref[1])
        sv = (z0_ref[...] + z1_ref[...] + y1_ref[...]) * dinv + b_ref[...]
        h1 = jnp.where(sv >= 0.0, sv, sv * a_ref[...])
        y2_ref[...] = jnp.dot(h1, w_ref[...],
                              preferred_element_type=jnp.float32) * dinv

    return pl.pallas_call(
        body,
        grid=(m // bm,),
        in_specs=[
            pl.BlockSpec((bm, h), lambda i: (i, 0)),
            pl.BlockSpec((bm, h), lambda i: (i, 0)),
            pl.BlockSpec((bm, h), lambda i: (i, 0)),
            pl.BlockSpec((2, bm, 16), lambda i: (0, i, 0)),
            pl.BlockSpec((h, h2), lambda i: (0, 0)),
            pl.BlockSpec((1, h), lambda i: (0, 0)),
            pl.BlockSpec((1, 1), lambda i: (0, 0)),
        ],
        out_specs=pl.BlockSpec((bm, h2), lambda i: (i, 0)),
        out_shape=jax.ShapeDtypeStruct((m, h2), jnp.float32),
    )(z0, z1, y1, degP, W2, b1.reshape(1, h), a.reshape(1, 1))


def _tc_layer2_post(z0, z1, y2, degP, b2, fcW, fcb, a, bm=1000):
    m, h2 = y2.shape
    od = fcW.shape[1]

    def body(z0_ref, z1_ref, y2_ref, dp_ref, b_ref, w_ref, fb_ref, a_ref,
             f_ref, l_ref):
        dinv = _dinv_block(dp_ref[0], dp_ref[1])
        tv = (z0_ref[...] + z1_ref[...] + y2_ref[...]) * dinv + b_ref[...]
        f2 = jnp.where(tv >= 0.0, tv, tv * a_ref[...])
        f_ref[...] = f2
        l_ref[...] = jnp.dot(f2, w_ref[...],
                             preferred_element_type=jnp.float32) + fb_ref[...]

    return pl.pallas_call(
        body,
        grid=(m // bm,),
        in_specs=[
            pl.BlockSpec((bm, h2), lambda i: (i, 0)),
            pl.BlockSpec((bm, h2), lambda i: (i, 0)),
            pl.BlockSpec((bm, h2), lambda i: (i, 0)),
            pl.BlockSpec((2, bm, 16), lambda i: (0, i, 0)),
            pl.BlockSpec((1, h2), lambda i: (0, 0)),
            pl.BlockSpec((h2, od), lambda i: (0, 0)),
            pl.BlockSpec((1, od), lambda i: (0, 0)),
            pl.BlockSpec((1, 1), lambda i: (0, 0)),
        ],
        out_specs=[
            pl.BlockSpec((bm, h2), lambda i: (i, 0)),
            pl.BlockSpec((bm, od), lambda i: (i, 0)),
        ],
        out_shape=[
            jax.ShapeDtypeStruct((m, h2), jnp.float32),
            jax.ShapeDtypeStruct((m, od), jnp.float32),
        ],
    )(z0, z1, y2, degP, b2.reshape(1, h2), fcW, fcb.reshape(1, od),
      a.reshape(1, 1))


def kernel(x, edge_index, W1, b1, W2, b2, fcW, fcb, a):
    src = edge_index[0].astype(jnp.int32).reshape(NW, NCHUNK, K)
    dst = edge_index[1].astype(jnp.int32).reshape(NW, NCHUNK, K)

    degP = _sc_degree(dst)[:, :N_NODES]
    y1 = _tc_layer1_pre(x, W1, degP)
    z1 = _sc_aggregate(src, dst, y1, y1.shape[1])[:, :N_NODES]
    y2 = _tc_layer1_post(z1[0], z1[1], y1, degP, W2, b1, a)
    z2 = _sc_aggregate(src, dst, y2, y2.shape[1])[:, :N_NODES]
    feat2, logits = _tc_layer2_post(z2[0], z2[1], y2, degP, b2, fcW, fcb, a)
    return (feat2, logits)


# speedup vs baseline: 35.6240x; 1.3969x over previous
import functools

import jax
import jax.numpy as jnp
from jax import lax
from jax.experimental import pallas as pl
from jax.experimental.pallas import tpu as pltpu
from jax.experimental.pallas import tpu_sc as plsc

N_NODES = 10000
N_EDGES = 320000
NC = 2
NS = 16
NW = NC * NS
EPW = N_EDGES // NW
K = 80
NCHUNK = EPW // K
NPAD = 10240
RPT = NPAD // NS


def _zero_rows(buf, nrows, ncol16):
    def body(r, _):
        for j in range(ncol16):
            buf[r, pl.ds(j * 16, 16)] = jnp.zeros((16,), jnp.float32)
        return 0
    lax.fori_loop(0, nrows, body, 0)


def _copy_slab(src_buf, dst_ref, base):
    for kk in range(RPT // K):
        pltpu.sync_copy(src_buf, dst_ref.at[pl.ds(base + kk * K, K)])


def _sc_degree(dst3):
    mesh = plsc.VectorSubcoreMesh(core_axis_name="c", subcore_axis_name="s")

    @functools.partial(
        pl.kernel,
        out_type=jax.ShapeDtypeStruct((NC, NPAD, 16), jnp.float32),
        mesh=mesh,
        compiler_params=pltpu.CompilerParams(use_tc_tiling_on_sc=False),
        scratch_types=[
            pltpu.VMEM((NCHUNK, K), jnp.int32),
            pltpu.VMEM((K, 16), jnp.float32),
            pltpu.VMEM_SHARED((NPAD, 16), jnp.float32),
            pltpu.SemaphoreType.DMA,
        ],
    )
    def deg_kernel(dst_hbm, out_hbm, idx_v, buf_v, acc_sh, sem):
        c = lax.axis_index("c")
        s = lax.axis_index("s")
        wid = c * NS + s
        tbase = s * RPT
        _zero_rows(buf_v, K, 1)
        _copy_slab(buf_v, acc_sh, tbase)

        def ones_row(r, _):
            buf_v[r, :] = jnp.ones((16,), jnp.float32)
            return 0
        lax.fori_loop(0, K, ones_row, 0)
        pltpu.sync_copy(dst_hbm.at[wid], idx_v)
        plsc.subcore_barrier()

        def chunk(i, _):
            pltpu.async_copy(buf_v, acc_sh.at[idx_v.at[i]], sem, add=True)
            return 0
        lax.fori_loop(0, NCHUNK, chunk, 0)

        def drain(i, _):
            pltpu.make_async_copy(buf_v, acc_sh.at[idx_v.at[i]], sem).wait()
            return 0
        lax.fori_loop(0, NCHUNK, drain, 0)
        plsc.subcore_barrier()
        pltpu.sync_copy(acc_sh.at[pl.ds(tbase, RPT)],
                        out_hbm.at[c, pl.ds(tbase, RPT)])

    return deg_kernel(dst3)


def _sc_aggregate(src3, dst3, y, d):
    mesh = plsc.VectorSubcoreMesh(core_axis_name="c", subcore_axis_name="s")
    staged = d < 128
    nbuf = 2
    scratch = (
        [pltpu.VMEM((NCHUNK, K), jnp.int32),
         pltpu.VMEM((NCHUNK, K), jnp.int32)]
        + [pltpu.VMEM((K, d), jnp.float32) for _ in range(nbuf)]
        + ([pltpu.VMEM_SHARED((NPAD, d), jnp.float32)] if staged else [])
        + [pltpu.VMEM_SHARED((NPAD, d), jnp.float32)]
        + [pltpu.SemaphoreType.DMA for _ in range(2 * nbuf)]
    )
    if staged:
        y = jnp.concatenate(
            [y, jnp.zeros((NPAD - y.shape[0], d), jnp.float32)])

    @functools.partial(
        pl.kernel,
        out_type=jax.ShapeDtypeStruct((NC, NPAD, d), jnp.float32),
        mesh=mesh,
        compiler_params=pltpu.CompilerParams(use_tc_tiling_on_sc=False),
        scratch_types=scratch,
    )
    def agg_kernel(src_hbm, dst_hbm, y_hbm, out_hbm, src_v, dst_v, *rest):
        bufs = rest[:nbuf]
        rest = rest[nbuf:]
        if staged:
            y_sh = rest[0]
            rest = rest[1:]
        acc_sh = rest[0]
        gsem = rest[1:1 + nbuf]
        ssem = rest[1 + nbuf:1 + 2 * nbuf]
        c = lax.axis_index("c")
        s = lax.axis_index("s")
        wid = c * NS + s
        tbase = s * RPT
        _zero_rows(bufs[0], K, d // 16)
        _copy_slab(bufs[0], acc_sh, tbase)
        pltpu.sync_copy(src_hbm.at[wid], src_v)
        pltpu.sync_copy(dst_hbm.at[wid], dst_v)
        if staged:
            pltpu.sync_copy(y_hbm.at[pl.ds(tbase, RPT)],
                            y_sh.at[pl.ds(tbase, RPT)])
            table = y_sh
        else:
            table = y_hbm
        plsc.subcore_barrier()

        def gather(b, i):
            pltpu.async_copy(table.at[src_v.at[i]], bufs[b], gsem[b])

        def gather_wait(b, i):
            pltpu.make_async_copy(table.at[src_v.at[i]], bufs[b],
                                  gsem[b]).wait()

        def scat(b, i):
            pltpu.async_copy(bufs[b], acc_sh.at[dst_v.at[i]], ssem[b],
                             add=True)

        def scat_wait(b, i):
            pltpu.make_async_copy(bufs[b], acc_sh.at[dst_v.at[i]],
                                  ssem[b]).wait()

        dd = 1
        nvisit = NCHUNK + dd
        nstep = (nvisit + nbuf - 1) // nbuf

        def step(g, _):
            for b in range(nbuf):
                i = g * nbuf + b

                @pl.when(jnp.logical_and(i >= nbuf, i < NCHUNK))
                def _():
                    scat_wait(b, i - nbuf)

                @pl.when(i < NCHUNK)
                def _():
                    gather(b, i)

                j = i - dd
                bj = (b - dd) % nbuf

                @pl.when(jnp.logical_and(j >= 0, j < NCHUNK))
                def _():
                    gather_wait(bj, j)
                    scat(bj, j)
            return 0
        lax.fori_loop(0, nstep, step, 0)
        for j in range(NCHUNK - nbuf, NCHUNK):
            scat_wait(j % nbuf, j)
        plsc.subcore_barrier()
        pltpu.sync_copy(acc_sh.at[pl.ds(tbase, RPT)],
                        out_hbm.at[c, pl.ds(tbase, RPT)])

    return agg_kernel(src3, dst3, y)


def _dinv_block(dp0, dp1):
    return lax.rsqrt(dp0[:, 0:1] + dp1[:, 0:1] + 1.0)


def _tc_layer1_pre(x, W1, degP, bm=1000):
    m, din = x.shape
    h = W1.shape[1]

    def body(x_ref, w_ref, dp_ref, y_ref):
        dinv = _dinv_block(dp_ref[0], dp_ref[1])
        xw = jnp.dot(x_ref[...], w_ref[...],
                     preferred_element_type=jnp.float32)
        y_ref[...] = xw * dinv

    return pl.pallas_call(
        body,
        grid=(m // bm,),
        in_specs=[
            pl.BlockSpec((bm, din), lambda i: (i, 0)),
            pl.BlockSpec((din, h), lambda i: (0, 0)),
            pl.BlockSpec((2, bm, 16), lambda i: (0, i, 0)),
        ],
        out_specs=pl.BlockSpec((bm, h), lambda i: (i, 0)),
        out_shape=jax.ShapeDtypeStruct((m, h), jnp.float32),
    )(x, W1, degP)


def _tc_layer1_post(z0, z1, y1, degP, W2, b1, a, bm=1000):
    m, h = y1.shape
    h2 = W2.shape[1]

    def body(z0_ref, z1_ref, y1_ref, dp_ref, w_ref, b_ref, a_ref, y2_ref):
        dinv = _dinv_block(dp_ref[0], dp_ref[1])
        sv = (z0_ref[...] + z1_ref[...] + y1_ref[...]) * dinv + b_ref[...]
        h1 = jnp.where(sv >= 0.0, sv, sv * a_ref[...])
        y2_ref[...] = jnp.dot(h1, w_ref[...],
                              preferred_element_type=jnp.float32) * dinv

    return pl.pallas_call(
        body,
        grid=(m // bm,),
        in_specs=[
            pl.BlockSpec((bm, h), lambda i: (i, 0)),
            pl.BlockSpec((bm, h), lambda i: (i, 0)),
            pl.BlockSpec((bm, h), lambda i: (i, 0)),
            pl.BlockSpec((2, bm, 16), lambda i: (0, i, 0)),
            pl.BlockSpec((h, h2), lambda i: (0, 0)),
            pl.BlockSpec((1, h), lambda i: (0, 0)),
            pl.BlockSpec((1, 1), lambda i: (0, 0)),
        ],
        out_specs=pl.BlockSpec((bm, h2), lambda i: (i, 0)),
        out_shape=jax.ShapeDtypeStruct((m, h2), jnp.float32),
    )(z0, z1, y1, degP, W2, b1.reshape(1, h), a.reshape(1, 1))


def _tc_layer2_post(z0, z1, y2, degP, b2, fcW, fcb, a, bm=1000):
    m, h2 = y2.shape
    od = fcW.shape[1]

    def body(z0_ref, z1_ref, y2_ref, dp_ref, b_ref, w_ref, fb_ref, a_ref,
             f_ref, l_ref):
        dinv = _dinv_block(dp_ref[0], dp_ref[1])
        tv = (z0_ref[...] + z1_ref[...] + y2_ref[...]) * dinv + b_ref[...]
        f2 = jnp.where(tv >= 0.0, tv, tv * a_ref[...])
        f_ref[...] = f2
        l_ref[...] = jnp.dot(f2, w_ref[...],
                             preferred_element_type=jnp.float32) + fb_ref[...]

    return pl.pallas_call(
        body,
        grid=(m // bm,),
        in_specs=[
            pl.BlockSpec((bm, h2), lambda i: (i, 0)),
            pl.BlockSpec((bm, h2), lambda i: (i, 0)),
            pl.BlockSpec((bm, h2), lambda i: (i, 0)),
            pl.BlockSpec((2, bm, 16), lambda i: (0, i, 0)),
            pl.BlockSpec((1, h2), lambda i: (0, 0)),
            pl.BlockSpec((h2, od), lambda i: (0, 0)),
            pl.BlockSpec((1, od), lambda i: (0, 0)),
            pl.BlockSpec((1, 1), lambda i: (0, 0)),
        ],
        out_specs=[
            pl.BlockSpec((bm, h2), lambda i: (i, 0)),
            pl.BlockSpec((bm, od), lambda i: (i, 0)),
        ],
        out_shape=[
            jax.ShapeDtypeStruct((m, h2), jnp.float32),
            jax.ShapeDtypeStruct((m, od), jnp.float32),
        ],
    )(z0, z1, y2, degP, b2.reshape(1, h2), fcW, fcb.reshape(1, od),
      a.reshape(1, 1))


def kernel(x, edge_index, W1, b1, W2, b2, fcW, fcb, a):
    src = edge_index[0].astype(jnp.int32).reshape(NW, NCHUNK, K)
    dst = edge_index[1].astype(jnp.int32).reshape(NW, NCHUNK, K)

    degP = _sc_degree(dst)[:, :N_NODES]
    y1 = _tc_layer1_pre(x, W1, degP)
    z1 = _sc_aggregate(src, dst, y1, y1.shape[1])[:, :N_NODES]
    y2 = _tc_layer1_post(z1[0], z1[1], y1, degP, W2, b1, a)
    z2 = _sc_aggregate(src, dst, y2, y2.shape[1])[:, :N_NODES]
    feat2, logits = _tc_layer2_post(z2[0], z2[1], y2, degP, b2, fcW, fcb, a)
    return (feat2, logits)


# speedup vs baseline: 40.5034x; 1.1370x over previous
import functools

import jax
import jax.numpy as jnp
from jax import lax
from jax.experimental import pallas as pl
from jax.experimental.pallas import tpu as pltpu
from jax.experimental.pallas import tpu_sc as plsc

N_NODES = 10000
N_EDGES = 320000
NC = 2
NS = 16
NW = NC * NS
EPW = N_EDGES // NW
K = 80
NCHUNK = EPW // K
NPAD = 10240
RPT = NPAD // NS


def _zero_rows(buf, nrows, ncol16):
    def body(r, _):
        for j in range(ncol16):
            buf[r, pl.ds(j * 16, 16)] = jnp.zeros((16,), jnp.float32)
        return 0
    lax.fori_loop(0, nrows, body, 0)


def _copy_slab(src_buf, dst_ref, base, kc=K):
    for kk in range(RPT // kc):
        pltpu.sync_copy(src_buf, dst_ref.at[pl.ds(base + kk * kc, kc)])


def _sc_degree(dst3):
    mesh = plsc.VectorSubcoreMesh(core_axis_name="c", subcore_axis_name="s")

    @functools.partial(
        pl.kernel,
        out_type=jax.ShapeDtypeStruct((NC, NPAD, 16), jnp.float32),
        mesh=mesh,
        compiler_params=pltpu.CompilerParams(use_tc_tiling_on_sc=False),
        scratch_types=[
            pltpu.VMEM((NCHUNK, K), jnp.int32),
            pltpu.VMEM((K, 16), jnp.float32),
            pltpu.VMEM_SHARED((NPAD, 16), jnp.float32),
            pltpu.SemaphoreType.DMA,
        ],
    )
    def deg_kernel(dst_hbm, out_hbm, idx_v, buf_v, acc_sh, sem):
        c = lax.axis_index("c")
        s = lax.axis_index("s")
        wid = c * NS + s
        tbase = s * RPT
        _zero_rows(buf_v, K, 1)
        _copy_slab(buf_v, acc_sh, tbase)

        def ones_row(r, _):
            buf_v[r, :] = jnp.ones((16,), jnp.float32)
            return 0
        lax.fori_loop(0, K, ones_row, 0)
        pltpu.sync_copy(dst_hbm.at[wid], idx_v)
        plsc.subcore_barrier()

        def chunk(i, _):
            pltpu.async_copy(buf_v, acc_sh.at[idx_v.at[i]], sem, add=True)
            return 0
        lax.fori_loop(0, NCHUNK, chunk, 0)

        def drain(i, _):
            pltpu.make_async_copy(buf_v, acc_sh.at[idx_v.at[i]], sem).wait()
            return 0
        lax.fori_loop(0, NCHUNK, drain, 0)
        plsc.subcore_barrier()
        pltpu.sync_copy(acc_sh.at[pl.ds(tbase, RPT)],
                        out_hbm.at[c, pl.ds(tbase, RPT)])

    return deg_kernel(dst3)


def _sc_aggregate(src3, dst3, y, d, kc, nbuf, dd):
    mesh = plsc.VectorSubcoreMesh(core_axis_name="c", subcore_axis_name="s")
    staged = d < 128
    nchunk = EPW // kc
    scratch = (
        [pltpu.VMEM((nchunk, kc), jnp.int32),
         pltpu.VMEM((nchunk, kc), jnp.int32)]
        + [pltpu.VMEM((kc, d), jnp.float32) for _ in range(nbuf)]
        + ([pltpu.VMEM_SHARED((NPAD, d), jnp.float32)] if staged else [])
        + [pltpu.VMEM_SHARED((NPAD, d), jnp.float32)]
        + [pltpu.SemaphoreType.DMA for _ in range(2 * nbuf)]
    )

    @functools.partial(
        pl.kernel,
        out_type=jax.ShapeDtypeStruct((NC, NPAD, d), jnp.float32),
        mesh=mesh,
        compiler_params=pltpu.CompilerParams(use_tc_tiling_on_sc=False),
        scratch_types=scratch,
    )
    def agg_kernel(src_hbm, dst_hbm, y_hbm, out_hbm, src_v, dst_v, *rest):
        bufs = rest[:nbuf]
        rest = rest[nbuf:]
        if staged:
            y_sh = rest[0]
            rest = rest[1:]
        acc_sh = rest[0]
        gsem = rest[1:1 + nbuf]
        ssem = rest[1 + nbuf:1 + 2 * nbuf]
        c = lax.axis_index("c")
        s = lax.axis_index("s")
        wid = c * NS + s
        tbase = s * RPT
        _zero_rows(bufs[0], kc, d // 16)
        _copy_slab(bufs[0], acc_sh, tbase, kc)
        pltpu.sync_copy(src_hbm.at[wid], src_v)
        pltpu.sync_copy(dst_hbm.at[wid], dst_v)
        if staged:
            ty = jnp.minimum(tbase, N_NODES - RPT)
            pltpu.sync_copy(y_hbm.at[pl.ds(ty, RPT)],
                            y_sh.at[pl.ds(ty, RPT)])
            table = y_sh
        else:
            table = y_hbm
        plsc.subcore_barrier()

        def gather(b, i):
            pltpu.async_copy(table.at[src_v.at[i]], bufs[b], gsem[b])

        def gather_wait(b, i):
            pltpu.make_async_copy(table.at[src_v.at[i]], bufs[b],
                                  gsem[b]).wait()

        def scat(b, i):
            pltpu.async_copy(bufs[b], acc_sh.at[dst_v.at[i]], ssem[b],
                             add=True)

        def scat_wait(b, i):
            pltpu.make_async_copy(bufs[b], acc_sh.at[dst_v.at[i]],
                                  ssem[b]).wait()

        nvisit = nchunk + dd
        nstep = (nvisit + nbuf - 1) // nbuf

        def step(g, _):
            for b in range(nbuf):
                i = g * nbuf + b

                @pl.when(jnp.logical_and(i >= nbuf, i < nchunk))
                def _():
                    scat_wait(b, i - nbuf)

                @pl.when(i < nchunk)
                def _():
                    gather(b, i)

                j = i - dd
                bj = (b - dd) % nbuf

                @pl.when(jnp.logical_and(j >= 0, j < nchunk))
                def _():
                    gather_wait(bj, j)
                    scat(bj, j)
            return 0
        lax.fori_loop(0, nstep, step, 0)
        for j in range(nchunk - nbuf, nchunk):
            scat_wait(j % nbuf, j)
        plsc.subcore_barrier()
        pltpu.sync_copy(acc_sh.at[pl.ds(tbase, RPT)],
                        out_hbm.at[c, pl.ds(tbase, RPT)])

    return agg_kernel(src3, dst3, y)


def _dinv_block(dp0, dp1):
    return lax.rsqrt(dp0[:, 0:1] + dp1[:, 0:1] + 1.0)


def _tc_layer1_pre(x, W1, degP, bm=1000):
    m, din = x.shape
    h = W1.shape[1]

    def body(x_ref, w_ref, dp_ref, y_ref):
        dinv = _dinv_block(dp_ref[0], dp_ref[1])
        xw = jnp.dot(x_ref[...], w_ref[...],
                     preferred_element_type=jnp.float32)
        y_ref[...] = xw * dinv

    return pl.pallas_call(
        body,
        grid=(m // bm,),
        in_specs=[
            pl.BlockSpec((bm, din), lambda i: (i, 0)),
            pl.BlockSpec((din, h), lambda i: (0, 0)),
            pl.BlockSpec((2, bm, 16), lambda i: (0, i, 0)),
        ],
        out_specs=pl.BlockSpec((bm, h), lambda i: (i, 0)),
        out_shape=jax.ShapeDtypeStruct((m, h), jnp.float32),
    )(x, W1, degP)


def _tc_layer1_post(z, y1, degP, W2, b1, a, bm=1000):
    m, h = y1.shape
    h2 = W2.shape[1]

    def body(z0_ref, z1_ref, y1_ref, dp_ref, w_ref, b_ref, a_ref, y2_ref):
        dinv = _dinv_block(dp_ref[0], dp_ref[1])
        sv = (z0_ref[0] + z1_ref[0] + y1_ref[...]) * dinv + b_ref[...]
        h1 = jnp.where(sv >= 0.0, sv, sv * a_ref[...])
        y2_ref[...] = jnp.dot(h1, w_ref[...],
                              preferred_element_type=jnp.float32) * dinv

    return pl.pallas_call(
        body,
        grid=(m // bm,),
        in_specs=[
            pl.BlockSpec((1, bm, h), lambda i: (0, i, 0)),
            pl.BlockSpec((1, bm, h), lambda i: (1, i, 0)),
            pl.BlockSpec((bm, h), lambda i: (i, 0)),
            pl.BlockSpec((2, bm, 16), lambda i: (0, i, 0)),
            pl.BlockSpec((h, h2), lambda i: (0, 0)),
            pl.BlockSpec((1, h), lambda i: (0, 0)),
            pl.BlockSpec((1, 1), lambda i: (0, 0)),
        ],
        out_specs=pl.BlockSpec((bm, h2), lambda i: (i, 0)),
        out_shape=jax.ShapeDtypeStruct((m, h2), jnp.float32),
    )(z, z, y1, degP, W2, b1.reshape(1, h), a.reshape(1, 1))


def _tc_layer2_post(z, y2, degP, b2, fcW, fcb, a, bm=1000):
    m, h2 = y2.shape
    od = fcW.shape[1]

    def body(z0_ref, z1_ref, y2_ref, dp_ref, b_ref, w_ref, fb_ref, a_ref,
             f_ref, l_ref):
        dinv = _dinv_block(dp_ref[0], dp_ref[1])
        tv = (z0_ref[0] + z1_ref[0] + y2_ref[...]) * dinv + b_ref[...]
        f2 = jnp.where(tv >= 0.0, tv, tv * a_ref[...])
        f_ref[...] = f2
        l_ref[...] = jnp.dot(f2, w_ref[...],
                             preferred_element_type=jnp.float32) + fb_ref[...]

    return pl.pallas_call(
        body,
        grid=(m // bm,),
        in_specs=[
            pl.BlockSpec((1, bm, h2), lambda i: (0, i, 0)),
            pl.BlockSpec((1, bm, h2), lambda i: (1, i, 0)),
            pl.BlockSpec((bm, h2), lambda i: (i, 0)),
            pl.BlockSpec((2, bm, 16), lambda i: (0, i, 0)),
            pl.BlockSpec((1, h2), lambda i: (0, 0)),
            pl.BlockSpec((h2, od), lambda i: (0, 0)),
            pl.BlockSpec((1, od), lambda i: (0, 0)),
            pl.BlockSpec((1, 1), lambda i: (0, 0)),
        ],
        out_specs=[
            pl.BlockSpec((bm, h2), lambda i: (i, 0)),
            pl.BlockSpec((bm, od), lambda i: (i, 0)),
        ],
        out_shape=[
            jax.ShapeDtypeStruct((m, h2), jnp.float32),
            jax.ShapeDtypeStruct((m, od), jnp.float32),
        ],
    )(z, z, y2, degP, b2.reshape(1, h2), fcW, fcb.reshape(1, od),
      a.reshape(1, 1))


def kernel(x, edge_index, W1, b1, W2, b2, fcW, fcb, a):
    src = edge_index[0].astype(jnp.int32)
    dst = edge_index[1].astype(jnp.int32)
    src80 = src.reshape(NW, NCHUNK, K)
    dst80 = dst.reshape(NW, NCHUNK, K)
    src40 = src.reshape(NW, EPW // 40, 40)
    dst40 = dst.reshape(NW, EPW // 40, 40)

    degP = _sc_degree(dst80)
    y1 = _tc_layer1_pre(x, W1, degP)
    z1 = _sc_aggregate(src40, dst40, y1, 128, kc=40, nbuf=4, dd=2)
    y2 = _tc_layer1_post(z1, y1, degP, W2, b1, a)
    z2 = _sc_aggregate(src80, dst80, y2, 16, kc=80, nbuf=2, dd=1)
    feat2, logits = _tc_layer2_post(z2, y2, degP, b2, fcW, fcb, a)
    return (feat2, logits)


# speedup vs baseline: 41.0732x; 1.0141x over previous
import functools

import jax
import jax.numpy as jnp
from jax import lax
from jax.experimental import pallas as pl
from jax.experimental.pallas import tpu as pltpu
from jax.experimental.pallas import tpu_sc as plsc

N_NODES = 10000
N_EDGES = 320000
NC = 2
NS = 16
NW = NC * NS
EPW = N_EDGES // NW
K = 80
NCHUNK = EPW // K
NPAD = 10112
RPT = NPAD // NS


def _zero_rows(buf, nrows, ncol16):
    def body(r, _):
        for j in range(ncol16):
            buf[r, pl.ds(j * 16, 16)] = jnp.zeros((16,), jnp.float32)
        return 0
    lax.fori_loop(0, nrows, body, 0)


def _copy_slab(src_buf, dst_ref, base, kc=K):
    for kk in range(RPT // kc):
        pltpu.sync_copy(src_buf, dst_ref.at[pl.ds(base + kk * kc, kc)])
    rem = RPT % kc
    if rem:
        pltpu.sync_copy(src_buf.at[pl.ds(0, rem)],
                        dst_ref.at[pl.ds(base + RPT - rem, rem)])


def _sc_degree(dst3):
    mesh = plsc.VectorSubcoreMesh(core_axis_name="c", subcore_axis_name="s")

    @functools.partial(
        pl.kernel,
        out_type=jax.ShapeDtypeStruct((NC, NPAD, 16), jnp.float32),
        mesh=mesh,
        compiler_params=pltpu.CompilerParams(use_tc_tiling_on_sc=False),
        scratch_types=[
            pltpu.VMEM((NCHUNK, K), jnp.int32),
            pltpu.VMEM((K, 16), jnp.float32),
            pltpu.VMEM_SHARED((NPAD, 16), jnp.float32),
            pltpu.SemaphoreType.DMA,
        ],
    )
    def deg_kernel(dst_hbm, out_hbm, idx_v, buf_v, acc_sh, sem):
        c = lax.axis_index("c")
        s = lax.axis_index("s")
        wid = c * NS + s
        tbase = s * RPT
        _zero_rows(buf_v, K, 1)
        _copy_slab(buf_v, acc_sh, tbase)

        def ones_row(r, _):
            buf_v[r, :] = jnp.ones((16,), jnp.float32)
            return 0
        lax.fori_loop(0, K, ones_row, 0)
        pltpu.sync_copy(dst_hbm.at[wid], idx_v)
        plsc.subcore_barrier()

        def chunk(i, _):
            pltpu.async_copy(buf_v, acc_sh.at[idx_v.at[i]], sem, add=True)
            return 0
        lax.fori_loop(0, NCHUNK, chunk, 0)

        def drain(i, _):
            pltpu.make_async_copy(buf_v, acc_sh.at[idx_v.at[i]], sem).wait()
            return 0
        lax.fori_loop(0, NCHUNK, drain, 0)
        plsc.subcore_barrier()
        pltpu.sync_copy(acc_sh.at[pl.ds(tbase, RPT)],
                        out_hbm.at[c, pl.ds(tbase, RPT)])

    return deg_kernel(dst3)


def _sc_aggregate(src3, dst3, y, d, kc, nbuf, dd):
    mesh = plsc.VectorSubcoreMesh(core_axis_name="c", subcore_axis_name="s")
    staged = d < 128
    nchunk = EPW // kc
    scratch = (
        [pltpu.VMEM((nchunk, kc), jnp.int32),
         pltpu.VMEM((nchunk, kc), jnp.int32)]
        + [pltpu.VMEM((kc, d), jnp.float32) for _ in range(nbuf)]
        + ([pltpu.VMEM_SHARED((NPAD, d), jnp.float32)] if staged else [])
        + [pltpu.VMEM_SHARED((NPAD, d), jnp.float32)]
        + [pltpu.SemaphoreType.DMA for _ in range(2 * nbuf)]
    )

    @functools.partial(
        pl.kernel,
        out_type=jax.ShapeDtypeStruct((NC, NPAD, d), jnp.float32),
        mesh=mesh,
        compiler_params=pltpu.CompilerParams(use_tc_tiling_on_sc=False),
        scratch_types=scratch,
    )
    def agg_kernel(src_hbm, dst_hbm, y_hbm, out_hbm, src_v, dst_v, *rest):
        bufs = rest[:nbuf]
        rest = rest[nbuf:]
        if staged:
            y_sh = rest[0]
            rest = rest[1:]
        acc_sh = rest[0]
        gsem = rest[1:1 + nbuf]
        ssem = rest[1 + nbuf:1 + 2 * nbuf]
        c = lax.axis_index("c")
        s = lax.axis_index("s")
        wid = c * NS + s
        tbase = s * RPT
        _zero_rows(bufs[0], kc, d // 16)
        _copy_slab(bufs[0], acc_sh, tbase, kc)
        pltpu.sync_copy(src_hbm.at[wid], src_v)
        pltpu.sync_copy(dst_hbm.at[wid], dst_v)
        if staged:
            ty = jnp.minimum(tbase, N_NODES - RPT)
            pltpu.sync_copy(y_hbm.at[pl.ds(ty, RPT)],
                            y_sh.at[pl.ds(ty, RPT)])
            table = y_sh
        else:
            table = y_hbm
        plsc.subcore_barrier()

        def gather(b, i):
            pltpu.async_copy(table.at[src_v.at[i]], bufs[b], gsem[b])

        def gather_wait(b, i):
            pltpu.make_async_copy(table.at[src_v.at[i]], bufs[b],
                                  gsem[b]).wait()

        def scat(b, i):
            pltpu.async_copy(bufs[b], acc_sh.at[dst_v.at[i]], ssem[b],
                             add=True)

        def scat_wait(b, i):
            pltpu.make_async_copy(bufs[b], acc_sh.at[dst_v.at[i]],
                                  ssem[b]).wait()

        nvisit = nchunk + dd
        nstep = (nvisit + nbuf - 1) // nbuf

        def step(g, _):
            for b in range(nbuf):
                i = g * nbuf + b

                @pl.when(jnp.logical_and(i >= nbuf, i < nchunk))
                def _():
                    scat_wait(b, i - nbuf)

                @pl.when(i < nchunk)
                def _():
                    gather(b, i)

                j = i - dd
                bj = (b - dd) % nbuf

                @pl.when(jnp.logical_and(j >= 0, j < nchunk))
                def _():
                    gather_wait(bj, j)
                    scat(bj, j)
            return 0
        lax.fori_loop(0, nstep, step, 0)
        for j in range(nchunk - nbuf, nchunk):
            scat_wait(j % nbuf, j)
        plsc.subcore_barrier()
        pltpu.sync_copy(acc_sh.at[pl.ds(tbase, RPT)],
                        out_hbm.at[c, pl.ds(tbase, RPT)])

    return agg_kernel(src3, dst3, y)


def _dinv_block(dp0, dp1):
    return lax.rsqrt(dp0[:, 0:1] + dp1[:, 0:1] + 1.0)


def _tc_layer1_pre(x, W1, degP, bm=1000):
    m, din = x.shape
    h = W1.shape[1]

    def body(x_ref, w_ref, dp_ref, y_ref):
        dinv = _dinv_block(dp_ref[0], dp_ref[1])
        xw = jnp.dot(x_ref[...], w_ref[...],
                     preferred_element_type=jnp.float32)
        y_ref[...] = xw * dinv

    return pl.pallas_call(
        body,
        grid=(m // bm,),
        in_specs=[
            pl.BlockSpec((bm, din), lambda i: (i, 0)),
            pl.BlockSpec((din, h), lambda i: (0, 0)),
            pl.BlockSpec((2, bm, 16), lambda i: (0, i, 0)),
        ],
        out_specs=pl.BlockSpec((bm, h), lambda i: (i, 0)),
        out_shape=jax.ShapeDtypeStruct((m, h), jnp.float32),
    )(x, W1, degP)


def _tc_layer1_post(z, y1, degP, W2, b1, a, bm=1000):
    m, h = y1.shape
    h2 = W2.shape[1]

    def body(z0_ref, z1_ref, y1_ref, dp_ref, w_ref, b_ref, a_ref, y2_ref):
        dinv = _dinv_block(dp_ref[0], dp_ref[1])
        sv = (z0_ref[0] + z1_ref[0] + y1_ref[...]) * dinv + b_ref[...]
        h1 = jnp.where(sv >= 0.0, sv, sv * a_ref[...])
        y2_ref[...] = jnp.dot(h1, w_ref[...],
                              preferred_element_type=jnp.float32) * dinv

    return pl.pallas_call(
        body,
        grid=(m // bm,),
        in_specs=[
            pl.BlockSpec((1, bm, h), lambda i: (0, i, 0)),
            pl.BlockSpec((1, bm, h), lambda i: (1, i, 0)),
            pl.BlockSpec((bm, h), lambda i: (i, 0)),
            pl.BlockSpec((2, bm, 16), lambda i: (0, i, 0)),
            pl.BlockSpec((h, h2), lambda i: (0, 0)),
            pl.BlockSpec((1, h), lambda i: (0, 0)),
            pl.BlockSpec((1, 1), lambda i: (0, 0)),
        ],
        out_specs=pl.BlockSpec((bm, h2), lambda i: (i, 0)),
        out_shape=jax.ShapeDtypeStruct((m, h2), jnp.float32),
    )(z, z, y1, degP, W2, b1.reshape(1, h), a.reshape(1, 1))


def _tc_layer2_post(z, y2, degP, b2, fcW, fcb, a, bm=1000):
    m, h2 = y2.shape
    od = fcW.shape[1]

    def body(z0_ref, z1_ref, y2_ref, dp_ref, b_ref, w_ref, fb_ref, a_ref,
             f_ref, l_ref):
        dinv = _dinv_block(dp_ref[0], dp_ref[1])
        tv = (z0_ref[0] + z1_ref[0] + y2_ref[...]) * dinv + b_ref[...]
        f2 = jnp.where(tv >= 0.0, tv, tv * a_ref[...])
        f_ref[...] = f2
        l_ref[...] = jnp.dot(f2, w_ref[...],
                             preferred_element_type=jnp.float32) + fb_ref[...]

    return pl.pallas_call(
        body,
        grid=(m // bm,),
        in_specs=[
            pl.BlockSpec((1, bm, h2), lambda i: (0, i, 0)),
            pl.BlockSpec((1, bm, h2), lambda i: (1, i, 0)),
            pl.BlockSpec((bm, h2), lambda i: (i, 0)),
            pl.BlockSpec((2, bm, 16), lambda i: (0, i, 0)),
            pl.BlockSpec((1, h2), lambda i: (0, 0)),
            pl.BlockSpec((h2, od), lambda i: (0, 0)),
            pl.BlockSpec((1, od), lambda i: (0, 0)),
            pl.BlockSpec((1, 1), lambda i: (0, 0)),
        ],
        out_specs=[
            pl.BlockSpec((bm, h2), lambda i: (i, 0)),
            pl.BlockSpec((bm, od), lambda i: (i, 0)),
        ],
        out_shape=[
            jax.ShapeDtypeStruct((m, h2), jnp.float32),
            jax.ShapeDtypeStruct((m, od), jnp.float32),
        ],
    )(z, z, y2, degP, b2.reshape(1, h2), fcW, fcb.reshape(1, od),
      a.reshape(1, 1))


def kernel(x, edge_index, W1, b1, W2, b2, fcW, fcb, a):
    src = edge_index[0].astype(jnp.int32)
    dst = edge_index[1].astype(jnp.int32)
    src80 = src.reshape(NW, NCHUNK, K)
    dst80 = dst.reshape(NW, NCHUNK, K)
    src40 = src.reshape(NW, EPW // 40, 40)
    dst40 = dst.reshape(NW, EPW // 40, 40)

    degP = _sc_degree(dst80)
    y1 = _tc_layer1_pre(x, W1, degP)
    z1 = _sc_aggregate(src40, dst40, y1, 128, kc=40, nbuf=5, dd=2)
    y2 = _tc_layer1_post(z1, y1, degP, W2, b1, a)
    z2 = _sc_aggregate(src40, dst40, y2, 16, kc=40, nbuf=4, dd=2)
    feat2, logits = _tc_layer2_post(z2, y2, degP, b2, fcW, fcb, a)
    return (feat2, logits)
